# baseline (speedup 1.0000x reference)
import jax
import jax.numpy as jnp
from jax import lax
from jax.experimental import pallas as pl
from jax.experimental.pallas import tpu as pltpu
from jax.experimental.pallas import tpu_sc as plsc

VOCAB = 1000000
D = 64
ROWS = 16384
COLS = 50
NC = 2
NS = 16
NW = NC * NS
IPW = ROWS // NW
IBL = 4
L = 16


def _emb_body(xt_hbm, table_hbm, out_hbm, idx_v, idx2_v, g0, g1, g2, g3,
              t0, t1, t2, t3, sg0, sg1, sg2, sg3, st0, st1, st2, st3):
    wid = lax.axis_index("s") * NC + lax.axis_index("c")
    gs = (g0, g1, g2, g3)
    ts = (t0, t1, t2, t3)
    sgs = (sg0, sg1, sg2, sg3)
    sts = (st0, st1, st2, st3)

    pltpu.sync_copy(xt_hbm.at[:, pl.ds(wid * IPW, IPW)], idx_v)

    iota = lax.iota(jnp.int32, L)

    def prep_idx2(j, ibl):
        for c in range(8):
            v = idx_v[j, pl.ds(128 * ibl + L * c, L)]
            idx2_v[ibl, pl.ds(L * c, L)] = lax.shift_right_logical(v, 1)

    def fire(j, ibl):
        prep_idx2(j, ibl)
        pltpu.async_copy(table_hbm.at[idx2_v.at[ibl]], gs[ibl], sgs[ibl])

    def drain_gather(ibl):
        pltpu.make_async_copy(
            table_hbm.at[idx2_v.at[ibl]], gs[ibl], sgs[ibl]).wait()

    def transpose(j, ibl):
        g, t = gs[ibl], ts[ibl]
        offs = []
        rows = []
        for c in range(8):
            v = idx_v[j, pl.ds(128 * ibl + L * c, L)]
            offs.append(lax.shift_left(lax.bitwise_and(v, 1), 6))
            rows.append(iota + L * c)

        def db_body(db, _):
            for ds in range(8):
                dd = 8 * db + ds
                for c in range(8):
                    vec = plsc.load_gather(g, [rows[c], offs[c] + dd])
                    t[db, ds, pl.ds(L * c, L)] = vec
            return ()

        lax.fori_loop(0, 8, db_body, (), unroll=False)

    def store(j, ibl):
        ibg = IBL * wid + ibl
        pltpu.async_copy(ts[ibl], out_hbm.at[j, :, ibg], sts[ibl])

    def wait_store(j, ibl):
        ibg = IBL * wid + ibl
        pltpu.make_async_copy(ts[ibl], out_hbm.at[j, :, ibg], sts[ibl]).wait()

    for ibl in range(IBL):
        fire(0, ibl)

    def j_body(j, _):
        for ibl in range(IBL):
            drain_gather(ibl)

            @pl.when(j > 0)
            def _():
                wait_store(j - 1, ibl)

            transpose(j, ibl)
            store(j, ibl)

            @pl.when(j < COLS - 1)
            def _():
                fire(j + 1, ibl)
        return ()

    lax.fori_loop(0, COLS, j_body, (), unroll=False)

    for ibl in range(IBL):
        wait_store(COLS - 1, ibl)


@jax.jit
def _emb_call(x, weight):
    mesh = plsc.VectorSubcoreMesh(core_axis_name="c", subcore_axis_name="s")
    xt = x.T
    w2 = weight.reshape(VOCAB // 2, 2 * D)
    p5 = pl.kernel(
        _emb_body,
        out_type=jax.ShapeDtypeStruct((COLS, 8, 128, 8, 128), jnp.float32),
        mesh=mesh,
        scratch_types=[
            pltpu.VMEM((COLS, IPW), jnp.int32),
            pltpu.VMEM((IBL, 128), jnp.int32),
            pltpu.VMEM((128, 128), jnp.float32),
            pltpu.VMEM((128, 128), jnp.float32),
            pltpu.VMEM((128, 128), jnp.float32),
            pltpu.VMEM((128, 128), jnp.float32),
            pltpu.VMEM((8, 8, 128), jnp.float32),
            pltpu.VMEM((8, 8, 128), jnp.float32),
            pltpu.VMEM((8, 8, 128), jnp.float32),
            pltpu.VMEM((8, 8, 128), jnp.float32),
            pltpu.SemaphoreType.DMA,
            pltpu.SemaphoreType.DMA,
            pltpu.SemaphoreType.DMA,
            pltpu.SemaphoreType.DMA,
            pltpu.SemaphoreType.DMA,
            pltpu.SemaphoreType.DMA,
            pltpu.SemaphoreType.DMA,
            pltpu.SemaphoreType.DMA,
        ],
        compiler_params=pltpu.CompilerParams(
            use_tc_tiling_on_sc=True, needs_layout_passes=False),
    )(xt, w2)
    out6 = p5.transpose(2, 4, 0, 1, 3)
    return out6.reshape(ROWS, COLS, D)


def kernel(x, weight):
    return _emb_call(x.astype(jnp.int32), weight)


# speedup vs baseline: 1.4565x; 1.4565x over previous
import jax
import jax.numpy as jnp
from jax import lax
from jax.experimental import pallas as pl
from jax.experimental.pallas import tpu as pltpu
from jax.experimental.pallas import tpu_sc as plsc

VOCAB = 1000000
D = 64
ROWS = 16384
COLS = 50
NC = 2
NS = 16
NW = NC * NS
ROWS_PER_W = ROWS // NW
GROUP = 8
GROUPS = ROWS_PER_W // GROUP


def _fire(table_hbm, idx_v, rows_buf, sem, g):
    for j in range(GROUP):
        pltpu.async_copy(
            table_hbm.at[idx_v.at[g * GROUP + j]],
            rows_buf.at[j],
            sem,
        )


def _drain(table_hbm, idx_v, rows_buf, sem):
    for j in range(GROUP):
        pltpu.make_async_copy(
            table_hbm.at[idx_v.at[j]],
            rows_buf.at[j],
            sem,
        ).wait()


def _emb_body(x_hbm, table_hbm, out_hbm, idx_v, rows0, rows1, sem0, sem1):
    wid = lax.axis_index("s") * NC + lax.axis_index("c")
    row_base = wid * ROWS_PER_W
    pltpu.sync_copy(x_hbm.at[pl.ds(row_base, ROWS_PER_W)], idx_v)

    def store(rows_buf, g):
        pltpu.sync_copy(rows_buf, out_hbm.at[pl.ds(row_base + g * GROUP, GROUP)])

    _fire(table_hbm, idx_v, rows0, sem0, 0)

    def pair_body(i, _):
        g = 2 * i
        _drain(table_hbm, idx_v, rows0, sem0)
        _fire(table_hbm, idx_v, rows1, sem1, g + 1)
        store(rows0, g)
        _drain(table_hbm, idx_v, rows1, sem1)
        _fire(table_hbm, idx_v, rows0, sem0, g + 2)
        store(rows1, g + 1)
        return ()

    lax.fori_loop(0, GROUPS // 2 - 1, pair_body, (), unroll=False)

    g = GROUPS - 2
    _drain(table_hbm, idx_v, rows0, sem0)
    _fire(table_hbm, idx_v, rows1, sem1, g + 1)
    store(rows0, g)
    _drain(table_hbm, idx_v, rows1, sem1)
    store(rows1, g + 1)


@jax.jit
def _emb_call(x, weight):
    mesh = plsc.VectorSubcoreMesh(core_axis_name="c", subcore_axis_name="s")
    return pl.kernel(
        _emb_body,
        out_type=jax.ShapeDtypeStruct((ROWS, COLS, D), jnp.float32),
        mesh=mesh,
        scratch_types=[
            pltpu.VMEM((ROWS_PER_W, COLS), jnp.int32),
            pltpu.VMEM((GROUP, COLS, D), jnp.float32),
            pltpu.VMEM((GROUP, COLS, D), jnp.float32),
            pltpu.SemaphoreType.DMA,
            pltpu.SemaphoreType.DMA,
        ],
        compiler_params=pltpu.CompilerParams(use_tc_tiling_on_sc=False),
    )(x, weight)


def kernel(x, weight):
    return _emb_call(x.astype(jnp.int32), weight)
